# Initial kernel scaffold; baseline (speedup 1.0000x reference)
import jax
import jax.numpy as jnp
from jax.experimental import pallas as pl


def kernel(x, edge_index, W1l, b1, W1r, W2l, b2, W2r):
    raise NotImplementedError("write your pallas kernel here")


# speedup vs baseline: 6.3607x; 6.3607x over previous
import functools

import jax
import jax.numpy as jnp
from jax import lax
from jax.experimental import pallas as pl
from jax.experimental.pallas import tpu as pltpu
from jax.experimental.pallas import tpu_sc as plsc

N = 10000
E = 320000
D = 128
NC = 2
NS = 16
K = 128
CHUNKS = E // K
CH_PER_CORE = CHUNKS // NC
K_STEPS = -(-CH_PER_CORE // NS)
RPT = 624
REM = N - NS * RPT
NPAD = 10240
QPT = NPAD // NS


def _seg_sum_kernel(with_count: bool):
    mesh = plsc.VectorSubcoreMesh(core_axis_name="c", subcore_axis_name="s")
    out_type = [jax.ShapeDtypeStruct((NC, N, D), jnp.float32)]
    scratch = [
        pltpu.VMEM_SHARED((N, D), jnp.float32),
        pltpu.VMEM((K,), jnp.int32),
        pltpu.VMEM((1, K), jnp.int32),
        pltpu.VMEM((K, D), jnp.float32),
        pltpu.SemaphoreType.DMA,
    ]
    if with_count:
        out_type.append(jax.ShapeDtypeStruct((NC, NPAD), jnp.float32))
        scratch += [
            pltpu.VMEM_SHARED((NPAD,), jnp.float32),
            pltpu.VMEM((K,), jnp.float32),
            pltpu.VMEM((QPT,), jnp.float32),
        ]

    def body(feat, edges, zf, *rest):
        if with_count:
            (zc, ones_h, out, cnt_out, acc, src_v, dst_v, rows_v, sem,
             cacc, ones_v, cbuf) = rest
        else:
            out, acc, src_v, dst_v, rows_v, sem = rest
        c = lax.axis_index("c")
        w = lax.axis_index("s")
        r0 = w * RPT
        pltpu.sync_copy(zf.at[pl.ds(0, K)], rows_v)
        for j in range(RPT // K):
            pltpu.sync_copy(rows_v, acc.at[pl.ds(r0 + j * K, K)])
        tail = RPT % K
        pltpu.sync_copy(rows_v.at[pl.ds(0, tail)],
                        acc.at[pl.ds(r0 + RPT - tail, tail)])

        @pl.when(w == 0)
        def _():
            pltpu.sync_copy(rows_v.at[pl.ds(0, REM)],
                            acc.at[pl.ds(NS * RPT, REM)])
        if with_count:
            q0 = w * QPT
            pltpu.sync_copy(zc.at[pl.ds(q0, QPT)], cbuf)
            pltpu.sync_copy(cbuf, cacc.at[pl.ds(q0, QPT)])
            pltpu.sync_copy(ones_h, ones_v)
        plsc.subcore_barrier()

        base = c * CH_PER_CORE + w
        limit = (c + 1) * CH_PER_CORE

        def step(k, carry):
            chunk = base + k * NS

            @pl.when(chunk < limit)
            def _():
                off = chunk * K
                pltpu.sync_copy(edges.at[0, pl.ds(off, K)], src_v)
                pltpu.sync_copy(edges.at[1, pl.ds(off, K)], dst_v.at[0])
                pltpu.async_copy(feat.at[src_v], rows_v, sem).wait()
                didx = dst_v.at[0]
                pltpu.sync_copy(rows_v, acc.at[didx], add=True)
                if with_count:
                    pltpu.sync_copy(ones_v, cacc.at[didx], add=True)

            return carry

        lax.fori_loop(0, K_STEPS, step, 0)
        plsc.subcore_barrier()
        for j in range(RPT // K):
            pltpu.sync_copy(acc.at[pl.ds(r0 + j * K, K)], rows_v)
            pltpu.sync_copy(rows_v, out.at[c, pl.ds(r0 + j * K, K)])
        pltpu.sync_copy(acc.at[pl.ds(r0 + RPT - tail, tail)],
                        rows_v.at[pl.ds(0, tail)])
        pltpu.sync_copy(rows_v.at[pl.ds(0, tail)],
                        out.at[c, pl.ds(r0 + RPT - tail, tail)])

        @pl.when(w == 0)
        def _():
            pltpu.sync_copy(acc.at[pl.ds(NS * RPT, REM)],
                            rows_v.at[pl.ds(0, REM)])
            pltpu.sync_copy(rows_v.at[pl.ds(0, REM)],
                            out.at[c, pl.ds(NS * RPT, REM)])
        if with_count:
            pltpu.sync_copy(cacc.at[pl.ds(q0, QPT)], cbuf)
            pltpu.sync_copy(cbuf, cnt_out.at[c, pl.ds(q0, QPT)])

    out = out_type if with_count else out_type[0]
    return pl.kernel(body, out_type=out, mesh=mesh, scratch_types=scratch)


_seg_sum_cnt = _seg_sum_kernel(with_count=True)
_seg_sum = _seg_sum_kernel(with_count=False)

_BN = 1000


def _sage_combine(relu: bool):
    def body(parts_ref, cnt_ref, x_ref, wl_ref, wr_ref, b_ref, o_ref):
        s = parts_ref[0] + parts_ref[1]
        cnt1 = cnt_ref[0] + cnt_ref[1]
        mean = s / jnp.maximum(cnt1, 1.0)
        acc = jnp.dot(mean, wl_ref[...], preferred_element_type=jnp.float32)
        acc = acc + jnp.dot(x_ref[...], wr_ref[...],
                            preferred_element_type=jnp.float32)
        acc = acc + b_ref[...]
        o_ref[...] = jnp.maximum(acc, 0.0) if relu else acc

    return pl.pallas_call(
        body,
        grid=(N // _BN,),
        in_specs=[
            pl.BlockSpec((NC, _BN, D), lambda i: (0, i, 0)),
            pl.BlockSpec((NC, _BN, 1), lambda i: (0, i, 0)),
            pl.BlockSpec((_BN, D), lambda i: (i, 0)),
            pl.BlockSpec((D, D), lambda i: (0, 0)),
            pl.BlockSpec((D, D), lambda i: (0, 0)),
            pl.BlockSpec((1, D), lambda i: (0, 0)),
        ],
        out_specs=pl.BlockSpec((_BN, D), lambda i: (i, 0)),
        out_shape=jax.ShapeDtypeStruct((N, D), jnp.float32),
    )


_combine_relu = _sage_combine(relu=True)
_combine_lin = _sage_combine(relu=False)


def kernel(x, edge_index, W1l, b1, W1r, W2l, b2, W2r):
    zf = jnp.zeros((N, D), jnp.float32)
    zc = jnp.zeros((NPAD,), jnp.float32)
    ones = jnp.ones((K,), jnp.float32)
    parts1, cnt_p = _seg_sum_cnt(x, edge_index, zf, zc, ones)
    cnts = cnt_p[:, :N, None]
    h = _combine_relu(parts1, cnts, x, W1l, W1r, b1.reshape(1, D))
    parts2 = _seg_sum(h, edge_index, zf)
    return _combine_lin(parts2, cnts, h, W2l, W2r, b2.reshape(1, D))
